# Initial kernel scaffold; baseline (speedup 1.0000x reference)
import jax
import jax.numpy as jnp
from jax.experimental import pallas as pl


def kernel(z_e, dictionary):
    raise NotImplementedError("write your pallas kernel here")


# speedup vs baseline: 2.9178x; 2.9178x over previous
import jax
import jax.numpy as jnp
from jax.experimental import pallas as pl

NUM_EMB = 1024
EMB_DIM = 64
SPARSITY = 5
COMMIT = 0.25
EPS = 1e-10

B_TOK = 1024


def _omp_block_kernel(x_ref, d_ref, coef_ref, zst_ref, loss_ref):
    dict_raw = d_ref[...]
    norm = jnp.sqrt(jnp.sum(dict_raw * dict_raw, axis=0, keepdims=True))
    dn = dict_raw / norm

    x = x_ref[...]
    res = x
    coef = jnp.zeros((NUM_EMB, B_TOK), jnp.float32)
    row_iota = jax.lax.broadcasted_iota(jnp.int32, (NUM_EMB, B_TOK), 0)

    for _ in range(SPARSITY):
        corr = jax.lax.dot_general(
            dn, res, (((0,), (0,)), ((), ())),
            preferred_element_type=jnp.float32)
        acorr = jnp.abs(corr)
        mx = jnp.max(acorr, axis=0, keepdims=True)
        idx = jnp.min(jnp.where(acorr == mx, row_iota, NUM_EMB),
                      axis=0, keepdims=True)
        onehot = (row_iota == idx).astype(jnp.float32)
        d_sel = jax.lax.dot_general(
            dn, onehot, (((1,), (0,)), ((), ())),
            precision=jax.lax.Precision.HIGHEST,
            preferred_element_type=jnp.float32)
        num = jnp.sum(res * d_sel, axis=0, keepdims=True)
        den = jnp.sum(d_sel * d_sel, axis=0, keepdims=True)
        alpha = num / (den + EPS)
        coef = coef + onehot * alpha
        res = res - d_sel * alpha

    coef_ref[...] = coef
    zdl = jax.lax.dot_general(
        dn, coef, (((1,), (0,)), ((), ())),
        preferred_element_type=jnp.float32)
    diff = zdl - x
    zst_ref[...] = x + diff

    s = jnp.sum(diff * diff).reshape(1, 1)
    i = pl.program_id(0)
    nblocks = pl.num_programs(0)

    @pl.when(i == 0)
    def _init():
        loss_ref[...] = s

    @pl.when(i != 0)
    def _acc():
        loss_ref[...] = loss_ref[...] + s

    @pl.when(i == nblocks - 1)
    def _finish():
        total = nblocks * EMB_DIM * B_TOK
        loss_ref[...] = loss_ref[...] * ((1.0 + COMMIT) / total)


def kernel(z_e, dictionary):
    z = jnp.transpose(z_e, (0, 2, 3, 1))
    input_shape = z.shape
    zf = z.reshape(EMB_DIM, -1)
    n_tok = zf.shape[1]
    grid = n_tok // B_TOK

    coef, zst, loss = pl.pallas_call(
        _omp_block_kernel,
        grid=(grid,),
        in_specs=[
            pl.BlockSpec((EMB_DIM, B_TOK), lambda i: (0, i)),
            pl.BlockSpec((EMB_DIM, NUM_EMB), lambda i: (0, 0)),
        ],
        out_specs=[
            pl.BlockSpec((NUM_EMB, B_TOK), lambda i: (0, i)),
            pl.BlockSpec((EMB_DIM, B_TOK), lambda i: (0, i)),
            pl.BlockSpec((1, 1), lambda i: (0, 0)),
        ],
        out_shape=[
            jax.ShapeDtypeStruct((NUM_EMB, n_tok), jnp.float32),
            jax.ShapeDtypeStruct((EMB_DIM, n_tok), jnp.float32),
            jax.ShapeDtypeStruct((1, 1), jnp.float32),
        ],
    )(zf, dictionary)

    z_st = jnp.transpose(zst.reshape(input_shape), (0, 3, 1, 2))
    return (z_st, loss[0, 0], coef)
